# baseline (speedup 1.0000x reference)
import functools
import math

import jax
import jax.numpy as jnp
from jax import lax
from jax.experimental import pallas as pl
from jax.experimental.pallas import tpu as pltpu
from jax.experimental.pallas import tpu_sc as plsc

VOCAB = 1000000
DIM = 64
B = 16384
L = 20

NC = 2
NS = 16
NW = NC * NS

SENT_PER_W = B // NW
TOK_PER_W = SENT_PER_W * L
SENT_PER_CHUNK = 4
TOK_PER_CHUNK = SENT_PER_CHUNK * L
N_CHUNKS = SENT_PER_W // SENT_PER_CHUNK

INV_SQRT_L = 1.0 / math.sqrt(float(L))

NBUF = 2


def _sc_body(ids_hbm, table_hbm, out_hbm, ids_v, idxhi_v, rows_v, out_v, sems):
  wid = lax.axis_index("s") * NC + lax.axis_index("c")

  pltpu.sync_copy(ids_hbm.at[pl.ds(wid * TOK_PER_W, TOK_PER_W)], ids_v)

  def start_gather(j, b):
    base = pl.multiple_of(j * TOK_PER_CHUNK, 16)
    for k in range(TOK_PER_CHUNK // 16):
      ids16 = ids_v[pl.ds(base + k * 16, 16)]
      idxhi_v[b, pl.ds(k * 16, 16)] = lax.shift_right_logical(ids16, 1)
    pltpu.async_copy(table_hbm.at[idxhi_v.at[b]], rows_v.at[b], sems.at[b])

  def accumulate(j, b):
    for s in range(SENT_PER_CHUNK):
      acc = [None] * (DIM // 16)
      for l in range(L):
        t = s * L + l
        idv = plsc.load_gather(
            ids_v, [jnp.zeros((16,), jnp.int32) + (j * TOK_PER_CHUNK + t)])
        p = (idv & 1).astype(jnp.float32)
        for d in range(DIM // 16):
          lo = rows_v[b, t, pl.ds(d * 16, 16)]
          hi = rows_v[b, t, pl.ds(64 + d * 16, 16)]
          val = lo + (hi - lo) * p
          acc[d] = val if l == 0 else acc[d] + val
      for d in range(DIM // 16):
        out_v[j * SENT_PER_CHUNK + s, pl.ds(d * 16, 16)] = acc[d] * INV_SQRT_L

  def wait(b):
    pltpu.make_async_copy(
        table_hbm.at[pl.ds(0, TOK_PER_CHUNK)], rows_v.at[b], sems.at[b]
    ).wait()

  for b in range(NBUF):
    start_gather(b, b)

  def ring_body(j):
    for b in range(NBUF):
      wait(b)
      accumulate(j + b, b)
      start_gather(j + b + NBUF, b)

  pl.loop(0, N_CHUNKS - NBUF, step=NBUF)(ring_body)

  for b in range(NBUF):
    wait(b)
    accumulate(N_CHUNKS - NBUF + b, b)

  pltpu.sync_copy(out_v, out_hbm.at[pl.ds(wid * SENT_PER_W, SENT_PER_W)])


@jax.jit
def _pooled_embedding(ids, table2):
  mesh = plsc.VectorSubcoreMesh(core_axis_name="c", subcore_axis_name="s")
  kern = functools.partial(
      pl.kernel,
      mesh=mesh,
      out_type=jax.ShapeDtypeStruct((B, DIM), jnp.float32),
      scratch_types=[
          pltpu.VMEM((TOK_PER_W,), jnp.int32),
          pltpu.VMEM((NBUF, TOK_PER_CHUNK), jnp.int32),
          pltpu.VMEM((NBUF, TOK_PER_CHUNK, 2 * DIM), jnp.float32),
          pltpu.VMEM((SENT_PER_W, DIM), jnp.float32),
          pltpu.SemaphoreType.DMA((NBUF,)),
      ],
      compiler_params=pltpu.CompilerParams(needs_layout_passes=False),
  )(_sc_body)
  return kern(ids, table2)


def kernel(token_ids, embedding_table):
  ids = token_ids.reshape(B * L)
  table2 = embedding_table.reshape(VOCAB // 2, 2 * DIM)
  return _pooled_embedding(ids, table2)
